# baseline (speedup 1.0000x reference)
import functools

import jax
import jax.numpy as jnp
from jax import lax
from jax.experimental import pallas as pl
from jax.experimental.pallas import tpu as pltpu
from jax.experimental.pallas import tpu_sc as plsc

_EPS2 = 1e-16
_MAGIC = 0x5F3759DF


def _rsqrt(m):
    i = plsc.bitcast(m, jnp.int32)
    y = plsc.bitcast(_MAGIC - (i >> 1), jnp.float32)
    for _ in range(3):
        y = y * (1.5 - 0.5 * m * y * y)
    return y


def _make_sc_call(B):
    info = plsc.get_sparse_core_info()
    NC, NS, L = info.num_cores, info.num_subcores, info.num_lanes
    NW = NC * NS
    chunk = B // NW

    mesh = plsc.VectorSubcoreMesh(core_axis_name="c", subcore_axis_name="s")
    f32, i32 = jnp.float32, jnp.int32

    @functools.partial(
        pl.kernel,
        mesh=mesh,
        out_type=jax.ShapeDtypeStruct((B,), f32),
        scratch_types=[pltpu.VMEM((chunk,), i32) for _ in range(8)]
        + [pltpu.VMEM((48, 512), f32),
           pltpu.VMEM((4,), f32), pltpu.VMEM((4,), f32), pltpu.VMEM((1,), f32),
           pltpu.VMEM((16,), f32), pltpu.VMEM((16, 16), f32),
           pltpu.VMEM((32,), f32), pltpu.VMEM((16, 32), f32),
           pltpu.MemorySpace.VMEM_SHARED((16, 16), f32),
           pltpu.MemorySpace.VMEM_SHARED((16, 32), f32),
           pltpu.VMEM((chunk,), f32), pltpu.SemaphoreType.DMA],
        compiler_params=pltpu.CompilerParams(needs_layout_passes=False,
                                             use_tc_tiling_on_sc=False),
    )
    def sc(ga, sa, ca, ma, gb, sb, cb, mb, gw, cw, sw, mw, wv, fcw, fcb, out,
           ga_v, sa_v, ca_v, ma_v, gb_v, sb_v, cb_v, mb_v,
           raw_v, wv_v, fcw_v, fcb_v, npub_v, nall_v, ppub_v, tbl_v,
           shared_n, shared_t, out_v, sem):
        s = lax.axis_index("s")
        cx = lax.axis_index("c")
        wid = s * NC + cx
        base = wid * chunk
        sl_h = pl.ds(base, chunk)
        zero = jnp.zeros((L,), f32)
        lane = lax.broadcasted_iota(i32, (L,), 0)

        copies = [
            pltpu.async_copy(ga.at[sl_h], ga_v, sem),
            pltpu.async_copy(sa.at[sl_h], sa_v, sem),
            pltpu.async_copy(ca.at[sl_h], ca_v, sem),
            pltpu.async_copy(ma.at[sl_h], ma_v, sem),
            pltpu.async_copy(gb.at[sl_h], gb_v, sem),
            pltpu.async_copy(sb.at[sl_h], sb_v, sem),
            pltpu.async_copy(cb.at[sl_h], cb_v, sem),
            pltpu.async_copy(mb.at[sl_h], mb_v, sem),
            pltpu.async_copy(gw, raw_v.at[pl.ds(0, 2), pl.ds(0, 4)], sem),
            pltpu.async_copy(cw, raw_v.at[pl.ds(8, 7), pl.ds(0, 64)], sem),
            pltpu.async_copy(sw, raw_v.at[pl.ds(16, 8), pl.ds(0, 64)], sem),
            pltpu.async_copy(mw, raw_v.at[pl.ds(24, 17), pl.ds(0, 512)], sem),
            pltpu.async_copy(wv, wv_v, sem),
            pltpu.async_copy(fcw, fcw_v, sem),
            pltpu.async_copy(fcb, fcb_v, sem),
        ]

        def zrow_g(r, carry):
            for ch in range(1, 32):
                raw_v[r, pl.ds(ch * L, L)] = zero
            return carry

        def zrow_cs(r, carry):
            for ch in range(4, 32):
                raw_v[r, pl.ds(ch * L, L)] = zero
            return carry

        lax.fori_loop(0, 2, zrow_g, 0)
        lax.fori_loop(8, 24, zrow_cs, 0)

        for cp in copies:
            cp.wait()
        for r in range(2):
            plsc.store_scatter(raw_v, [jnp.full((L,), r, i32), lane],
                               zero, mask=lane >= 4)

        def row_rn(r):
            acc = zero
            for ch in range(32):
                v = raw_v[r, pl.ds(ch * L, L)]
                acc = acc + v * v
            n2 = jnp.sum(acc)
            return _rsqrt(jnp.full((L,), jnp.maximum(n2, _EPS2), f32))

        rv1 = row_rn(3 * s)
        rv2 = row_rn(3 * s + 1)
        rv3 = row_rn(3 * s + 2)
        npub_v[...] = jnp.where(lane == 0, rv1,
                                jnp.where(lane == 1, rv2,
                                          jnp.where(lane == 2, rv3, 0.0)))
        pltpu.sync_copy(npub_v, shared_n.at[s])
        plsc.subcore_barrier()
        pltpu.sync_copy(shared_n, nall_v)

        def rn_of(r):
            row = r // 3
            ln = r - row * 3
            return plsc.load_gather(nall_v, [jnp.full((L,), row, i32),
                                             jnp.full((L,), ln, i32)])

        zi = jnp.zeros((L,), i32)
        biasv = plsc.load_gather(fcb_v, [zi])

        def pair_body(k, res):
            res0, res1 = res
            e = jnp.minimum(s * 32 + k, 405)
            is_g = e < 4
            is_c = (e >= 4) & (e < 53)
            is_s = (e >= 53) & (e < 117)
            off = jnp.where(is_g, 0, jnp.where(is_c, 4, jnp.where(is_s, 53, 117)))
            ncol = jnp.where(is_g, 2, jnp.where(is_c, 7, jnp.where(is_s, 8, 17)))
            row0 = jnp.where(is_g, 0, jnp.where(is_c, 8, jnp.where(is_s, 16, 24)))
            kt = jnp.where(is_g, 0, jnp.where(is_c, 1, jnp.where(is_s, 2, 3)))
            loc = e - off
            i = loc // ncol
            j = loc - i * ncol
            rA = row0 + i
            rB = row0 + j
            acc = zero
            for ch in range(32):
                acc = acc + (raw_v[rA, pl.ds(ch * L, L)]
                             * raw_v[rB, pl.ds(ch * L, L)])
            dt = jnp.sum(acc)
            val = jnp.full((L,), dt, f32) * rn_of(rA) * rn_of(rB)
            ktv = jnp.full((L,), kt, i32)
            val = val * (plsc.load_gather(wv_v, [ktv])
                         * plsc.load_gather(fcw_v, [ktv]))
            val = val + jnp.where(is_g, biasv, zero)
            m = lane == (k & 15)
            res0 = jnp.where(m & (k < 16), val, res0)
            res1 = jnp.where(m & (k >= 16), val, res1)
            return (res0, res1)

        res0, res1 = lax.fori_loop(0, 32, pair_body, (zero, zero))
        ppub_v[pl.ds(0, L)] = res0
        ppub_v[pl.ds(L, L)] = res1
        pltpu.sync_copy(ppub_v, shared_t.at[s])
        plsc.subcore_barrier()
        pltpu.sync_copy(shared_t, tbl_v)

        def gat(e):
            return plsc.load_gather(tbl_v, [e >> 5, e & 31])

        for r in range(chunk // L):
            sl = pl.ds(r * L, L)
            v = (gat(ga_v[sl] * 2 + gb_v[sl])
                 + gat(ca_v[sl] * 7 + cb_v[sl] + 4)
                 + gat(sa_v[sl] * 8 + sb_v[sl] + 53)
                 + gat(ma_v[sl] * 17 + mb_v[sl] + 117))
            out_v[sl] = 1.0 / (1.0 + jnp.exp(-v))
        pltpu.sync_copy(out_v, out.at[sl_h])

    return sc


def kernel(gA, sA, cA, mA, gB, sB, cB, mB,
           gender_W, college_W, school_W, mbti_W, weight, fc_W, fc_b):
    B = gA.shape[0]
    i32 = jnp.int32
    out = _make_sc_call(B)(
        gA.astype(i32), sA.astype(i32), cA.astype(i32), mA.astype(i32),
        gB.astype(i32), sB.astype(i32), cB.astype(i32), mB.astype(i32),
        gender_W, college_W, school_W, mbti_W,
        weight, fc_W.reshape(-1), fc_b)
    return out.reshape(B, 1)
